# baseline (speedup 1.0000x reference)
import functools
import math

import jax
import jax.numpy as jnp
import numpy as np
from jax import lax
from jax.experimental import pallas as pl
from jax.experimental.pallas import tpu as pltpu
from jax.experimental.pallas import tpu_sc as plsc

N = 1
D = 256
H = 8
L = 4
P = 4
DH = D // H
SPATIAL = [(128, 128), (64, 64), (32, 32), (16, 16)]
LEN_IN = sum(h * w for h, w in SPATIAL)
LQ = LEN_IN
STARTS = np.cumsum([0] + [h * w for h, w in SPATIAL])[:-1].tolist()

BQ = 1280
NBLK = LQ // BQ

_K = np.arange(128)
_H_OF_K = _K // 16
_L_OF_K = (_K // 4) % 4
_P_OF_K = _K % 4
_AW_PERM = (_H_OF_K * 16 + _P_OF_K * 4 + _L_OF_K).tolist()

_VPERM = np.empty(D, np.int64)
for _h in range(H):
    for _i in range(16):
        _VPERM[_h * 32 + 2 * _i] = _h * 32 + _i
        _VPERM[_h * 32 + 2 * _i + 1] = _h * 32 + 16 + _i
_VPERM = _VPERM.tolist()


def _mmb_kernel(a_ref, bT_ref, bias_ref, o_ref):
    o_ref[...] = (
        jnp.dot(a_ref[...], bT_ref[...], preferred_element_type=jnp.float32)
        + bias_ref[...]
    ).astype(o_ref.dtype)


def _matmul_bias(a, w, b, out_dtype=jnp.float32):
    dout = w.shape[0]
    return pl.pallas_call(
        _mmb_kernel,
        grid=(NBLK,),
        in_specs=[
            pl.BlockSpec((BQ, D), lambda i: (i, 0)),
            pl.BlockSpec((D, dout), lambda i: (0, 0)),
            pl.BlockSpec((1, dout), lambda i: (0, 0)),
        ],
        out_specs=pl.BlockSpec((BQ, dout), lambda i: (i, 0)),
        out_shape=jax.ShapeDtypeStruct((a.shape[0], dout), out_dtype),
    )(a, w.T, b.reshape(1, dout))


def _prep_kernel(q_ref, refx_ref, refy_ref, woxT_ref, woyT_ref, waT_ref,
                 box_ref, boy_ref, ba_ref, wlf_ref, hlf_ref, wli_ref,
                 base_ref, idx_ref, w_ref):
    q = q_ref[...]
    offx = jnp.dot(q, woxT_ref[...], preferred_element_type=jnp.float32) + box_ref[...]
    offy = jnp.dot(q, woyT_ref[...], preferred_element_type=jnp.float32) + boy_ref[...]
    logits = jnp.dot(q, waT_ref[...], preferred_element_type=jnp.float32) + ba_ref[...]
    lg = logits.reshape(-1, H, 16)
    lg = lg - jnp.max(lg, axis=-1, keepdims=True)
    e = jnp.exp(lg)
    aw = (e / jnp.sum(e, axis=-1, keepdims=True)).reshape(-1, 128)

    wlf = wlf_ref[...]
    hlf = hlf_ref[...]
    x = refx_ref[...] * wlf + offx - 0.5
    y = refy_ref[...] * hlf + offy - 0.5
    x0 = jnp.floor(x)
    y0 = jnp.floor(y)
    wx1 = x - x0
    wx0 = 1.0 - wx1
    wy1 = y - y0
    wy0 = 1.0 - wy1
    wli = wli_ref[...]
    base = base_ref[...]
    for c, (dx, dy, wx, wy) in enumerate(
        [(0.0, 0.0, wx0, wy0), (1.0, 0.0, wx1, wy0),
         (0.0, 1.0, wx0, wy1), (1.0, 1.0, wx1, wy1)]):
        ix = x0 + dx
        iy = y0 + dy
        valid = ((ix >= 0.0) & (ix <= wlf - 1.0)
                 & (iy >= 0.0) & (iy <= hlf - 1.0))
        ixc = jnp.clip(ix, 0.0, wlf - 1.0).astype(jnp.int32)
        iyc = jnp.clip(iy, 0.0, hlf - 1.0).astype(jnp.int32)
        idx_ref[c] = base + (iyc * wli + ixc) * 8
        w_ref[c] = aw * (wy * wx) * valid.astype(jnp.float32)


def _prep(query2, refx_b, refy_b, woxT, woyT, waT, box, boy, ba,
          wlf, hlf, wli, base):
    spec128 = pl.BlockSpec((1, 128), lambda i: (0, 0))
    return pl.pallas_call(
        _prep_kernel,
        grid=(NBLK,),
        in_specs=[
            pl.BlockSpec((BQ, D), lambda i: (i, 0)),
            pl.BlockSpec((BQ, 128), lambda i: (i, 0)),
            pl.BlockSpec((BQ, 128), lambda i: (i, 0)),
            pl.BlockSpec((D, 128), lambda i: (0, 0)),
            pl.BlockSpec((D, 128), lambda i: (0, 0)),
            pl.BlockSpec((D, 128), lambda i: (0, 0)),
            spec128, spec128, spec128, spec128, spec128, spec128, spec128,
        ],
        out_specs=[
            pl.BlockSpec((4, BQ, 128), lambda i: (0, i, 0)),
            pl.BlockSpec((4, BQ, 128), lambda i: (0, i, 0)),
        ],
        out_shape=[
            jax.ShapeDtypeStruct((4, LQ, 128), jnp.int32),
            jax.ShapeDtypeStruct((4, LQ, 128), jnp.float32),
        ],
    )(query2, refx_b, refy_b, woxT, woyT, waT, box, boy, ba,
      wlf, hlf, wli, base)


NW = 32
QPT = LQ // NW
CQ = 2
NCH = QPT // CQ


def _sc_gather_body(idx_hbm, w_hbm, table_hbm, out_hbm,
                    idx_v, w_v, rows_v, out_v, sem):
    wid = lax.axis_index("s") * 2 + lax.axis_index("c")
    q0 = wid * QPT

    def chunk(ch, carry):
        qb = q0 + ch * CQ
        for c in range(4):
            pltpu.sync_copy(idx_hbm.at[c, pl.ds(qb, CQ)], idx_v.at[c])
            pltpu.sync_copy(w_hbm.at[c, pl.ds(qb, CQ)], w_v.at[c])
        copies = []
        for c in range(4):
            for qq in range(CQ):
                copies.append(pltpu.async_copy(
                    table_hbm.at[idx_v.at[c, qq]], rows_v.at[c, qq], sem))
        for cp in copies:
            cp.wait()

        def pair(pr, carry2):
            qq = pr // H
            h = pr % H
            a = [jnp.zeros((16,), jnp.float32) for _ in range(8)]
            for c in range(4):
                wv = w_v[c, qq, pl.ds(h * 16, 16)]
                for j in range(16):
                    r = h * 16 + j
                    ws = wv[j]
                    ev, od = plsc.unpack(
                        rows_v[c, qq, r, pl.ds(0, 32)],
                        format=plsc.PackFormat.INTERLEAVED,
                        preferred_element_type=jnp.float32)
                    a[2 * c] = a[2 * c] + ws * ev
                    a[2 * c + 1] = a[2 * c + 1] + ws * od
            out_v[pr, pl.ds(0, 16)] = (a[0] + a[2]) + (a[4] + a[6])
            out_v[pr, pl.ds(16, 16)] = (a[1] + a[3]) + (a[5] + a[7])
            return carry2

        lax.fori_loop(0, CQ * H, pair, 0)
        pltpu.sync_copy(out_v, out_hbm.at[pl.ds(qb * H, CQ * H)])
        return carry

    lax.fori_loop(0, NCH, chunk, 0)


def _sc_gather(idx, w, table):
    return pl.kernel(
        _sc_gather_body,
        out_type=jax.ShapeDtypeStruct((LQ * H, DH), jnp.float32),
        mesh=plsc.VectorSubcoreMesh(core_axis_name="c", subcore_axis_name="s"),
        scratch_types=[
            pltpu.VMEM((4, CQ, 128), jnp.int32),
            pltpu.VMEM((4, CQ, 128), jnp.float32),
            pltpu.VMEM((4, CQ, 128, DH), jnp.bfloat16),
            pltpu.VMEM((CQ * H, DH), jnp.float32),
            pltpu.SemaphoreType.DMA,
        ],
        compiler_params=pltpu.CompilerParams(
            use_tc_tiling_on_sc=False, needs_layout_passes=False),
    )(idx, w, table)


def kernel(query, reference_points, input_flatten, input_spatial_shapes,
           input_level_start_index, W_off, b_off, W_attn, b_attn,
           W_v, b_v, W_o, b_o):
    q2 = query[0]
    inf2 = input_flatten[0]

    l_of_k = jnp.asarray(_L_OF_K, jnp.int32)
    ssf = input_spatial_shapes.astype(jnp.float32)
    wlf = ssf[:, 1][l_of_k].reshape(1, 128)
    hlf = ssf[:, 0][l_of_k].reshape(1, 128)
    wli = input_spatial_shapes[:, 1][l_of_k].reshape(1, 128)
    base = (input_level_start_index[l_of_k] * 8
            + jnp.asarray(_H_OF_K, jnp.int32)).reshape(1, 128)

    woxT = W_off[0::2].T
    woyT = W_off[1::2].T
    box = b_off[0::2].reshape(1, 128)
    boy = b_off[1::2].reshape(1, 128)
    perm = jnp.asarray(_AW_PERM, jnp.int32)
    waT = W_attn[perm].T
    ba = b_attn[perm].reshape(1, 128)

    ref0 = reference_points[0]
    refx_b = ref0[:, :, 0][:, l_of_k]
    refy_b = ref0[:, :, 1][:, l_of_k]

    vperm = jnp.asarray(_VPERM, jnp.int32)
    value = _matmul_bias(inf2, W_v[vperm], b_v[vperm], jnp.bfloat16)
    table = value.reshape(LEN_IN * 8, DH)

    idx, w = _prep(q2, refx_b, refy_b, woxT, woyT, waT, box, boy, ba,
                   wlf, hlf, wli, base)

    attn_out = _sc_gather(idx, w, table).reshape(LQ, D)

    out = _matmul_bias(attn_out, W_o, b_o)
    return out.reshape(1, LQ, D)


# speedup vs baseline: 2.8524x; 2.8524x over previous
import functools
import math

import jax
import jax.numpy as jnp
import numpy as np
from jax import lax
from jax.experimental import pallas as pl
from jax.experimental.pallas import tpu as pltpu
from jax.experimental.pallas import tpu_sc as plsc

N = 1
D = 256
H = 8
L = 4
P = 4
DH = D // H
SPATIAL = [(128, 128), (64, 64), (32, 32), (16, 16)]
LEN_IN = sum(h * w for h, w in SPATIAL)
LQ = LEN_IN
STARTS = np.cumsum([0] + [h * w for h, w in SPATIAL])[:-1].tolist()

BQ = 1280
NBLK = LQ // BQ

_K = np.arange(128)
_H_OF_K = _K // 16
_L_OF_K = (_K // 4) % 4
_P_OF_K = _K % 4
_AW_PERM = (_H_OF_K * 16 + _P_OF_K * 4 + _L_OF_K).tolist()

_VPERM = np.empty(D, np.int64)
for _h in range(H):
    for _i in range(16):
        _VPERM[_h * 32 + 2 * _i] = _h * 32 + _i
        _VPERM[_h * 32 + 2 * _i + 1] = _h * 32 + 16 + _i
_VPERM = _VPERM.tolist()


def _mmb_kernel(a_ref, bT_ref, bias_ref, o_ref):
    o_ref[...] = (
        jnp.dot(a_ref[...], bT_ref[...], preferred_element_type=jnp.float32)
        + bias_ref[...]
    ).astype(o_ref.dtype)


def _matmul_bias(a, w, b, out_dtype=jnp.float32):
    dout = w.shape[0]
    return pl.pallas_call(
        _mmb_kernel,
        grid=(NBLK,),
        in_specs=[
            pl.BlockSpec((BQ, D), lambda i: (i, 0)),
            pl.BlockSpec((D, dout), lambda i: (0, 0)),
            pl.BlockSpec((1, dout), lambda i: (0, 0)),
        ],
        out_specs=pl.BlockSpec((BQ, dout), lambda i: (i, 0)),
        out_shape=jax.ShapeDtypeStruct((a.shape[0], dout), out_dtype),
    )(a, w.T, b.reshape(1, dout))


def _prep_kernel(q_ref, refx_ref, refy_ref, woxT_ref, woyT_ref, waT_ref,
                 box_ref, boy_ref, ba_ref, wlf_ref, hlf_ref, wli_ref,
                 base_ref, comb_ref):
    q = q_ref[...]
    offx = jnp.dot(q, woxT_ref[...], preferred_element_type=jnp.float32) + box_ref[...]
    offy = jnp.dot(q, woyT_ref[...], preferred_element_type=jnp.float32) + boy_ref[...]
    logits = jnp.dot(q, waT_ref[...], preferred_element_type=jnp.float32) + ba_ref[...]
    lg = logits.reshape(-1, H, 16)
    lg = lg - jnp.max(lg, axis=-1, keepdims=True)
    e = jnp.exp(lg)
    aw = (e / jnp.sum(e, axis=-1, keepdims=True)).reshape(-1, 128)

    wlf = wlf_ref[...]
    hlf = hlf_ref[...]
    x = refx_ref[...] * wlf + offx - 0.5
    y = refy_ref[...] * hlf + offy - 0.5
    x0 = jnp.floor(x)
    y0 = jnp.floor(y)
    wx1 = x - x0
    wx0 = 1.0 - wx1
    wy1 = y - y0
    wy0 = 1.0 - wy1
    wli = wli_ref[...]
    base = base_ref[...]
    for c, (dx, dy, wx, wy) in enumerate(
        [(0.0, 0.0, wx0, wy0), (1.0, 0.0, wx1, wy0),
         (0.0, 1.0, wx0, wy1), (1.0, 1.0, wx1, wy1)]):
        ix = x0 + dx
        iy = y0 + dy
        valid = ((ix >= 0.0) & (ix <= wlf - 1.0)
                 & (iy >= 0.0) & (iy <= hlf - 1.0))
        ixc = jnp.clip(ix, 0.0, wlf - 1.0).astype(jnp.int32)
        iyc = jnp.clip(iy, 0.0, hlf - 1.0).astype(jnp.int32)
        comb_ref[c] = base + (iyc * wli + ixc) * 8
        comb_ref[4 + c] = lax.bitcast_convert_type(
            aw * (wy * wx) * valid.astype(jnp.float32), jnp.int32)


def _prep(query2, refx_b, refy_b, woxT, woyT, waT, box, boy, ba,
          wlf, hlf, wli, base):
    spec128 = pl.BlockSpec((1, 128), lambda i: (0, 0))
    return pl.pallas_call(
        _prep_kernel,
        grid=(NBLK,),
        in_specs=[
            pl.BlockSpec((BQ, D), lambda i: (i, 0)),
            pl.BlockSpec((BQ, 128), lambda i: (i, 0)),
            pl.BlockSpec((BQ, 128), lambda i: (i, 0)),
            pl.BlockSpec((D, 128), lambda i: (0, 0)),
            pl.BlockSpec((D, 128), lambda i: (0, 0)),
            pl.BlockSpec((D, 128), lambda i: (0, 0)),
            spec128, spec128, spec128, spec128, spec128, spec128, spec128,
        ],
        out_specs=pl.BlockSpec((8, BQ, 128), lambda i: (0, i, 0)),
        out_shape=jax.ShapeDtypeStruct((8, LQ, 128), jnp.int32),
    )(query2, refx_b, refy_b, woxT, woyT, waT, box, boy, ba,
      wlf, hlf, wli, base)


NW = 32
QPT = LQ // NW
CQ = 4
NCH = QPT // CQ


def _sc_gather_body(comb_hbm, table_hbm, out_hbm,
                    slab_v, rows_v, out_v, sem0, sem1):
    wid = lax.axis_index("s") * 2 + lax.axis_index("c")
    q0 = wid * QPT
    sems = (sem0, sem1)

    def load_slab(buf, ch):
        qb = q0 + jnp.minimum(ch, NCH - 1) * CQ
        pltpu.sync_copy(comb_hbm.at[:, pl.ds(qb, CQ)], slab_v.at[buf])

    def fire(buf):
        for c in range(4):
            for cq in range(CQ):
                pltpu.async_copy(table_hbm.at[slab_v.at[buf, c, cq]],
                                 rows_v.at[buf, c, cq], sems[buf])

    def drain(buf):
        for c in range(4):
            for cq in range(CQ):
                pltpu.make_async_copy(table_hbm.at[slab_v.at[buf, c, cq]],
                                      rows_v.at[buf, c, cq], sems[buf]).wait()

    def accum(buf, ch):
        def pair(pr, carry2):
            qq = pr // H
            h = pr % H
            a = [jnp.zeros((16,), jnp.float32) for _ in range(8)]
            for c in range(4):
                wv = plsc.bitcast(
                    slab_v[buf, 4 + c, qq, pl.ds(h * 16, 16)], jnp.float32)
                for j in range(16):
                    r = h * 16 + j
                    ws = wv[j]
                    ev, od = plsc.unpack(
                        rows_v[buf, c, qq, r, pl.ds(0, 32)],
                        format=plsc.PackFormat.INTERLEAVED,
                        preferred_element_type=jnp.float32)
                    a[2 * c] = a[2 * c] + ws * ev
                    a[2 * c + 1] = a[2 * c + 1] + ws * od
            out_v[pr, pl.ds(0, 16)] = (a[0] + a[2]) + (a[4] + a[6])
            out_v[pr, pl.ds(16, 16)] = (a[1] + a[3]) + (a[5] + a[7])
            return carry2

        lax.fori_loop(0, CQ * H, pair, 0)
        qb = q0 + ch * CQ
        pltpu.sync_copy(out_v, out_hbm.at[pl.ds(qb * H, CQ * H)])

    load_slab(0, 0)
    load_slab(1, 1)
    fire(0)

    def step(g, carry):
        a_ch = 2 * g
        fire(1)
        drain(0)
        accum(0, a_ch)
        load_slab(0, a_ch + 2)
        fire(0)
        drain(1)
        accum(1, a_ch + 1)
        load_slab(1, a_ch + 3)
        return carry

    lax.fori_loop(0, NCH // 2, step, 0)
    drain(0)


def _sc_gather(comb, table):
    return pl.kernel(
        _sc_gather_body,
        out_type=jax.ShapeDtypeStruct((LQ * H, DH), jnp.float32),
        mesh=plsc.VectorSubcoreMesh(core_axis_name="c", subcore_axis_name="s"),
        scratch_types=[
            pltpu.VMEM((2, 8, CQ, 128), jnp.int32),
            pltpu.VMEM((2, 4, CQ, 128, DH), jnp.bfloat16),
            pltpu.VMEM((CQ * H, DH), jnp.float32),
            pltpu.SemaphoreType.DMA,
            pltpu.SemaphoreType.DMA,
        ],
        compiler_params=pltpu.CompilerParams(
            use_tc_tiling_on_sc=False, needs_layout_passes=False),
    )(comb, table)


def kernel(query, reference_points, input_flatten, input_spatial_shapes,
           input_level_start_index, W_off, b_off, W_attn, b_attn,
           W_v, b_v, W_o, b_o):
    q2 = query[0]
    inf2 = input_flatten[0]

    l_of_k = jnp.asarray(_L_OF_K, jnp.int32)
    ssf = input_spatial_shapes.astype(jnp.float32)
    wlf = ssf[:, 1][l_of_k].reshape(1, 128)
    hlf = ssf[:, 0][l_of_k].reshape(1, 128)
    wli = input_spatial_shapes[:, 1][l_of_k].reshape(1, 128)
    base = (input_level_start_index[l_of_k] * 8
            + jnp.asarray(_H_OF_K, jnp.int32)).reshape(1, 128)

    woxT = W_off[0::2].T
    woyT = W_off[1::2].T
    box = b_off[0::2].reshape(1, 128)
    boy = b_off[1::2].reshape(1, 128)
    perm = jnp.asarray(_AW_PERM, jnp.int32)
    waT = W_attn[perm].T
    ba = b_attn[perm].reshape(1, 128)

    ref0 = reference_points[0]
    refx_b = ref0[:, :, 0][:, l_of_k]
    refy_b = ref0[:, :, 1][:, l_of_k]

    vperm = jnp.asarray(_VPERM, jnp.int32)
    value = _matmul_bias(inf2, W_v[vperm], b_v[vperm], jnp.bfloat16)
    table = value.reshape(LEN_IN * 8, DH)

    comb = _prep(q2, refx_b, refy_b, woxT, woyT, waT, box, boy, ba,
                 wlf, hlf, wli, base)

    attn_out = _sc_gather(comb, table).reshape(LQ, D)

    out = _matmul_bias(attn_out, W_o, b_o)
    return out.reshape(1, LQ, D)
